# baseline (speedup 1.0000x reference)
import functools

import jax
import jax.numpy as jnp
from jax import lax
from jax.experimental import pallas as pl
from jax.experimental.pallas import tpu as pltpu
from jax.experimental.pallas import tpu_sc as plsc

_C = 64
_NBUF = 10


@functools.lru_cache(maxsize=None)
def _build_gather(batch: int, steps: int, D: int):
    info = plsc.get_sparse_core_info()
    nc, ns = info.num_cores, info.num_subcores
    nw = nc * ns
    wb = batch // nw
    S = wb // _C
    n_chunks = steps * S
    n_periods = n_chunks // _NBUF
    assert _C <= 128 and batch % nw == 0 and wb % _C == 0
    assert n_chunks % _NBUF == 0

    mesh = plsc.VectorSubcoreMesh(core_axis_name="c", subcore_axis_name="s")

    def body(table_hbm, idx_hbm, out_hbm, idx_v, rows_v, gsem, *wsems):
        wid = lax.axis_index("s") * nc + lax.axis_index("c")
        bbase = wid * wb
        pltpu.sync_copy(idx_hbm.at[wid], idx_v)

        def wait_writeout(b):
            pltpu.make_async_copy(
                rows_v.at[b], out_hbm.at[0, pl.ds(bbase, _C)], wsems[b]).wait()

        def out_slice(j):
            step = j // S
            h = j - step * S
            return out_hbm.at[step, pl.ds(bbase + h * _C, _C)]

        def period(o, carry):
            descs = []
            for b in range(_NBUF):
                j = o * _NBUF + b

                @pl.when(o > 0)
                def _():
                    wait_writeout(b)

                descs.append(pltpu.async_copy(
                    table_hbm.at[idx_v.at[j]], rows_v.at[b], gsem))
            for b in range(_NBUF):
                j = o * _NBUF + b
                descs[b].wait()
                pltpu.async_copy(rows_v.at[b], out_slice(j), wsems[b])
            return carry

        lax.fori_loop(0, n_periods, period, 0)
        for b in range(_NBUF):
            wait_writeout(b)

    return pl.kernel(
        body,
        mesh=mesh,
        out_type=jax.ShapeDtypeStruct((steps, batch, D), jnp.float32),
        scratch_types=[
            pltpu.VMEM((n_chunks, _C), jnp.int32),
            pltpu.VMEM((_NBUF, _C, D), jnp.float32),
            pltpu.SemaphoreType.DMA,
        ] + [pltpu.SemaphoreType.DMA] * _NBUF,
    )


def kernel(inputs, embedding):
    batch, steps = inputs.shape
    d = embedding.shape[1]
    info = plsc.get_sparse_core_info()
    nw = info.num_cores * info.num_subcores
    wb = batch // nw
    s = wb // _C
    idx = (inputs.astype(jnp.int32).T
           .reshape(steps, nw, s, _C)
           .transpose(1, 0, 2, 3)
           .reshape(nw, steps * s, _C))
    out = _build_gather(batch, steps, d)(embedding, idx)
    return out.transpose(1, 0, 2)


# speedup vs baseline: 1.0378x; 1.0378x over previous
import functools

import jax
import jax.numpy as jnp
from jax import lax
from jax.experimental import pallas as pl
from jax.experimental.pallas import tpu as pltpu
from jax.experimental.pallas import tpu_sc as plsc

_C = 128
_NBUF = 5


@functools.lru_cache(maxsize=None)
def _build_gather(batch: int, steps: int, D: int):
    info = plsc.get_sparse_core_info()
    nc, ns = info.num_cores, info.num_subcores
    nw = nc * ns
    wb = batch // nw
    S = wb // _C
    n_chunks = steps * S
    n_periods = n_chunks // _NBUF
    assert _C <= 128 and batch % nw == 0 and wb % _C == 0
    assert n_chunks % _NBUF == 0

    mesh = plsc.VectorSubcoreMesh(core_axis_name="c", subcore_axis_name="s")

    def body(table_hbm, idx_hbm, out_hbm, idx_v, rows_v, gsem, *wsems):
        wid = lax.axis_index("s") * nc + lax.axis_index("c")
        bbase = wid * wb
        pltpu.sync_copy(idx_hbm.at[wid], idx_v)

        def wait_writeout(b):
            pltpu.make_async_copy(
                rows_v.at[b], out_hbm.at[0, pl.ds(bbase, _C)], wsems[b]).wait()

        def out_slice(j):
            step = j // S
            h = j - step * S
            return out_hbm.at[step, pl.ds(bbase + h * _C, _C)]

        def period(o, carry):
            descs = []
            for b in range(_NBUF):
                j = o * _NBUF + b

                @pl.when(o > 0)
                def _():
                    wait_writeout(b)

                descs.append(pltpu.async_copy(
                    table_hbm.at[idx_v.at[j]], rows_v.at[b], gsem))
            for b in range(_NBUF):
                j = o * _NBUF + b
                descs[b].wait()
                pltpu.async_copy(rows_v.at[b], out_slice(j), wsems[b])
            return carry

        lax.fori_loop(0, n_periods, period, 0)
        for b in range(_NBUF):
            wait_writeout(b)

    return pl.kernel(
        body,
        mesh=mesh,
        out_type=jax.ShapeDtypeStruct((steps, batch, D), jnp.float32),
        scratch_types=[
            pltpu.VMEM((n_chunks, _C), jnp.int32),
            pltpu.VMEM((_NBUF, _C, D), jnp.float32),
            pltpu.SemaphoreType.DMA,
        ] + [pltpu.SemaphoreType.DMA] * _NBUF,
    )


def kernel(inputs, embedding):
    batch, steps = inputs.shape
    d = embedding.shape[1]
    info = plsc.get_sparse_core_info()
    nw = info.num_cores * info.num_subcores
    wb = batch // nw
    s = wb // _C
    idx = (inputs.astype(jnp.int32).T
           .reshape(steps, nw, s, _C)
           .transpose(1, 0, 2, 3)
           .reshape(nw, steps * s, _C))
    out = _build_gather(batch, steps, d)(embedding, idx)
    return out.transpose(1, 0, 2)


# speedup vs baseline: 1.0407x; 1.0028x over previous
import functools

import jax
import jax.numpy as jnp
from jax import lax
from jax.experimental import pallas as pl
from jax.experimental.pallas import tpu as pltpu
from jax.experimental.pallas import tpu_sc as plsc

_C = 128
_NBUF = 5


@functools.lru_cache(maxsize=None)
def _build_gather(batch: int, steps: int, D: int):
    info = plsc.get_sparse_core_info()
    nc, ns = info.num_cores, info.num_subcores
    nw = nc * ns
    wb = batch // nw
    S = wb // _C
    n_chunks = steps * S
    n_periods = n_chunks // _NBUF
    assert _C <= 128 and batch % nw == 0 and wb % _C == 0
    assert n_chunks % _NBUF == 0

    mesh = plsc.VectorSubcoreMesh(core_axis_name="c", subcore_axis_name="s")

    def body(table_hbm, idx_hbm, out_hbm, idx_v, rows_v, gsem, *wsems):
        wid = lax.axis_index("s") * nc + lax.axis_index("c")
        bbase = wid * wb
        pltpu.sync_copy(idx_hbm.at[wid], idx_v)

        def wait_writeout(b):
            pltpu.make_async_copy(
                rows_v.at[b], out_hbm.at[0, pl.ds(bbase, _C)], wsems[b]).wait()

        def out_slice(j):
            step = j // S
            h = j - step * S
            return out_hbm.at[step, pl.ds(bbase + h * _C, _C)]

        def period(o, carry):
            descs = []
            for b in range(_NBUF):
                j = o * _NBUF + b

                @pl.when(o > 0)
                def _():
                    wait_writeout(b)

                descs.append(pltpu.async_copy(
                    table_hbm.at[idx_v.at[j]], rows_v.at[b], gsem))
                if b >= 1:
                    descs[b - 1].wait()
                    pltpu.async_copy(
                        rows_v.at[b - 1], out_slice(j - 1), wsems[b - 1])
            descs[_NBUF - 1].wait()
            pltpu.async_copy(
                rows_v.at[_NBUF - 1],
                out_slice(o * _NBUF + _NBUF - 1), wsems[_NBUF - 1])
            return carry

        lax.fori_loop(0, n_periods, period, 0)
        for b in range(_NBUF):
            wait_writeout(b)

    return pl.kernel(
        body,
        mesh=mesh,
        out_type=jax.ShapeDtypeStruct((steps, batch, D), jnp.float32),
        scratch_types=[
            pltpu.VMEM((n_chunks, _C), jnp.int32),
            pltpu.VMEM((_NBUF, _C, D), jnp.float32),
            pltpu.SemaphoreType.DMA,
        ] + [pltpu.SemaphoreType.DMA] * _NBUF,
    )


def kernel(inputs, embedding):
    batch, steps = inputs.shape
    d = embedding.shape[1]
    info = plsc.get_sparse_core_info()
    nw = info.num_cores * info.num_subcores
    wb = batch // nw
    s = wb // _C
    idx = (inputs.astype(jnp.int32).T
           .reshape(steps, nw, s, _C)
           .transpose(1, 0, 2, 3)
           .reshape(nw, steps * s, _C))
    out = _build_gather(batch, steps, d)(embedding, idx)
    return out.transpose(1, 0, 2)
